# Initial kernel scaffold; baseline (speedup 1.0000x reference)
import functools

import jax
import jax.numpy as jnp
from jax import lax
from jax.experimental import pallas as pl
from jax.experimental.pallas import tpu as pltpu

N = 50000
E = 800000
HD = 32

NBLK = 2000
EBLK = 2000
N_GRID = N // NBLK
E_GRID = E // EBLK

F32 = jnp.float32


def _node_pre_body(x_ref, W_in_ref, b_in_ref, W1_ref, W2_ref, Wenc_ref,
                   W3_ref, benc_ref, bmsg_ref,
                   h_ref, A_ref, B_ref, Wp_ref, bp_ref):
    h = jnp.dot(x_ref[...], W_in_ref[...], preferred_element_type=F32) + b_in_ref[...]
    h_ref[...] = h
    A_ref[...] = jnp.dot(h, W1_ref[...], preferred_element_type=F32)
    B_ref[...] = jnp.dot(h, W2_ref[...], preferred_element_type=F32)

    @pl.when(pl.program_id(0) == 0)
    def _():
        Wp_ref[...] = jnp.dot(Wenc_ref[...], W3_ref[...], preferred_element_type=F32)
        bp_ref[...] = jnp.dot(benc_ref[...], W3_ref[...], preferred_element_type=F32) + bmsg_ref[...]


def _node_pre(x, W_in, b_in, W1, W2, W_enc, W3, b_enc, b_msg):
    const = pl.BlockSpec((HD, HD), lambda i: (0, 0))
    constb = pl.BlockSpec((1, HD), lambda i: (0, 0))
    return pl.pallas_call(
        _node_pre_body,
        grid=(N_GRID,),
        in_specs=[
            pl.BlockSpec((NBLK, 3), lambda i: (i, 0)),
            pl.BlockSpec((3, HD), lambda i: (0, 0)),
            constb, const, const, const, const, constb, constb,
        ],
        out_specs=[
            pl.BlockSpec((NBLK, HD), lambda i: (i, 0)),
            pl.BlockSpec((NBLK, HD), lambda i: (i, 0)),
            pl.BlockSpec((NBLK, HD), lambda i: (i, 0)),
            pl.BlockSpec((HD, HD), lambda i: (0, 0)),
            pl.BlockSpec((1, HD), lambda i: (0, 0)),
        ],
        out_shape=[
            jax.ShapeDtypeStruct((N, HD), F32),
            jax.ShapeDtypeStruct((N, HD), F32),
            jax.ShapeDtypeStruct((N, HD), F32),
            jax.ShapeDtypeStruct((HD, HD), F32),
            jax.ShapeDtypeStruct((1, HD), F32),
        ],
    )(x, W_in, b_in, W1, W2, W_enc, W3, b_enc, b_msg)


def _edge_T_body(hm_ref, Wp_ref, bp_ref, T_ref):
    T_ref[...] = jnp.dot(hm_ref[...], Wp_ref[...], preferred_element_type=F32) + bp_ref[...]


def _edge_T(h_msg, Wp, bp):
    return pl.pallas_call(
        _edge_T_body,
        grid=(E_GRID,),
        in_specs=[
            pl.BlockSpec((EBLK, HD), lambda i: (i, 0)),
            pl.BlockSpec((HD, HD), lambda i: (0, 0)),
            pl.BlockSpec((1, HD), lambda i: (0, 0)),
        ],
        out_specs=pl.BlockSpec((EBLK, HD), lambda i: (i, 0)),
        out_shape=jax.ShapeDtypeStruct((E, HD), F32),
    )(h_msg, Wp, bp)


def _y_msg_body(hm_ref, Wd_ref, bd_ref, y_ref):
    y_ref[...] = jnp.dot(hm_ref[...], Wd_ref[...], preferred_element_type=F32) + bd_ref[...]


def _y_msg(h_msg_new, W_dec, b_dec):
    return pl.pallas_call(
        _y_msg_body,
        grid=(E_GRID,),
        in_specs=[
            pl.BlockSpec((EBLK, HD), lambda i: (i, 0)),
            pl.BlockSpec((HD, 2), lambda i: (0, 0)),
            pl.BlockSpec((1, 2), lambda i: (0, 0)),
        ],
        out_specs=pl.BlockSpec((EBLK, 2), lambda i: (i, 0)),
        out_shape=jax.ShapeDtypeStruct((E, 2), F32),
    )(h_msg_new, W_dec, b_dec)


def _node_fin_body(h_ref, p0_ref, p1_ref, x_ref, Wu1_ref, Wu2_ref, bu_ref,
                   Wb1_ref, Wb2_ref, bb_ref, y_ref):
    agg = p0_ref[...] + p1_ref[...]
    hn = jnp.dot(h_ref[...], Wu1_ref[...], preferred_element_type=F32)
    hn = hn + jnp.dot(agg, Wu2_ref[...], preferred_element_type=F32) + bu_ref[...]
    hn = jnp.maximum(hn, 0.0)
    y = jnp.dot(hn, Wb1_ref[...], preferred_element_type=F32)
    y_ref[...] = y + jnp.dot(x_ref[...], Wb2_ref[...], preferred_element_type=F32) + bb_ref[...]


def _node_fin(h_node, p0, p1, x, Wu1, Wu2, bu, Wb1, Wb2, bb):
    const = pl.BlockSpec((HD, HD), lambda i: (0, 0))
    return pl.pallas_call(
        _node_fin_body,
        grid=(N_GRID,),
        in_specs=[
            pl.BlockSpec((NBLK, HD), lambda i: (i, 0)),
            pl.BlockSpec((NBLK, HD), lambda i: (i, 0)),
            pl.BlockSpec((NBLK, HD), lambda i: (i, 0)),
            pl.BlockSpec((NBLK, 3), lambda i: (i, 0)),
            const, const,
            pl.BlockSpec((1, HD), lambda i: (0, 0)),
            pl.BlockSpec((HD, 3), lambda i: (0, 0)),
            pl.BlockSpec((3, 3), lambda i: (0, 0)),
            pl.BlockSpec((1, 3), lambda i: (0, 0)),
        ],
        out_specs=pl.BlockSpec((NBLK, 3), lambda i: (i, 0)),
        out_shape=jax.ShapeDtypeStruct((N, 3), F32),
    )(h_node, p0, p1, x, Wu1, Wu2, bu, Wb1, Wb2, bb)


def kernel(x, edge_index, h_msg, W_in, b_in, W_enc, b_enc, W_msg, b_msg,
           W_upd, b_upd, W_dec, b_dec, W_bel, b_bel):
    W1 = W_msg[0:HD]
    W2 = W_msg[HD:2 * HD]
    W3 = W_msg[2 * HD:3 * HD]
    b_in2 = b_in.reshape(1, HD)
    b_enc2 = b_enc.reshape(1, HD)
    b_msg2 = b_msg.reshape(1, HD)
    b_upd2 = b_upd.reshape(1, HD)
    b_dec2 = b_dec.reshape(1, 2)
    b_bel2 = b_bel.reshape(1, 3)
    Wu1 = W_upd[0:HD]
    Wu2 = W_upd[HD:2 * HD]
    Wb1 = W_bel[0:HD]
    Wb2 = W_bel[HD:HD + 3]

    h_node, A, B, Wp, bp = _node_pre(x, W_in, b_in2, W1, W2, W_enc, W3,
                                     b_enc2, b_msg2)
    T = _edge_T(h_msg, Wp, bp)

    src = edge_index[0]
    dst = edge_index[1]
    h_msg_new = jnp.maximum(T + jnp.take(A, src, axis=0) + jnp.take(B, dst, axis=0), 0.0)
    agg = jax.ops.segment_sum(h_msg_new, dst, num_segments=N)
    p0 = agg
    p1 = jnp.zeros_like(agg)

    y_msg = _y_msg(h_msg_new, W_dec, b_dec2)
    y_beliefs = _node_fin(h_node, p0, p1, x, Wu1, Wu2, b_upd2, Wb1, Wb2, b_bel2)
    return (h_msg_new, y_msg, y_beliefs)


# speedup vs baseline: 3.0811x; 3.0811x over previous
import functools

import jax
import jax.numpy as jnp
from jax import lax
from jax.experimental import pallas as pl
from jax.experimental.pallas import tpu as pltpu
from jax.experimental.pallas import tpu_sc as plsc

N = 50000
E = 800000
HD = 32
Q = E // 4
G4 = 4

NBLK = 2000
EBLK = 2000
N_GRID = N // NBLK
Q_GRID = Q // EBLK

F32 = jnp.float32


def _node_pre_body(x_ref, W_in_ref, b_in_ref, W1_ref, W2_ref, Wenc_ref,
                   W3_ref, benc_ref, bmsg_ref,
                   h_ref, A_ref, B_ref, Wp_ref, bp_ref):
    h = jnp.dot(x_ref[...], W_in_ref[...], preferred_element_type=F32) + b_in_ref[...]
    h_ref[...] = h
    A_ref[...] = jnp.dot(h, W1_ref[...], preferred_element_type=F32)
    B_ref[...] = jnp.dot(h, W2_ref[...], preferred_element_type=F32)

    @pl.when(pl.program_id(0) == 0)
    def _():
        Wp_ref[...] = jnp.dot(Wenc_ref[...], W3_ref[...], preferred_element_type=F32)
        bp_ref[...] = jnp.dot(benc_ref[...], W3_ref[...], preferred_element_type=F32) + bmsg_ref[...]


def _node_pre(x, W_in, b_in, W1, W2, W_enc, W3, b_enc, b_msg):
    const = pl.BlockSpec((HD, HD), lambda i: (0, 0))
    constb = pl.BlockSpec((1, HD), lambda i: (0, 0))
    return pl.pallas_call(
        _node_pre_body,
        grid=(N_GRID,),
        in_specs=[
            pl.BlockSpec((NBLK, 3), lambda i: (i, 0)),
            pl.BlockSpec((3, HD), lambda i: (0, 0)),
            constb, const, const, const, const, constb, constb,
        ],
        out_specs=[
            pl.BlockSpec((NBLK, HD), lambda i: (i, 0)),
            pl.BlockSpec((NBLK, HD), lambda i: (i, 0)),
            pl.BlockSpec((NBLK, HD), lambda i: (i, 0)),
            pl.BlockSpec((HD, HD), lambda i: (0, 0)),
            pl.BlockSpec((1, HD), lambda i: (0, 0)),
        ],
        out_shape=[
            jax.ShapeDtypeStruct((N, HD), F32),
            jax.ShapeDtypeStruct((N, HD), F32),
            jax.ShapeDtypeStruct((N, HD), F32),
            jax.ShapeDtypeStruct((HD, HD), F32),
            jax.ShapeDtypeStruct((1, HD), F32),
        ],
    )(x, W_in, b_in, W1, W2, W_enc, W3, b_enc, b_msg)


def _edge_T_body(h0_ref, h1_ref, h2_ref, h3_ref, Wp_ref, bp_ref, T_ref):
    Wp = Wp_ref[...]
    bp = bp_ref[...]
    parts = [jnp.dot(h_ref[...], Wp, preferred_element_type=F32) + bp
             for h_ref in (h0_ref, h1_ref, h2_ref, h3_ref)]
    T_ref[...] = jnp.concatenate(parts, axis=1)


def _edge_T(h_msg, Wp, bp):
    def hmap(g):
        return lambda i: (g * Q_GRID + i, 0)
    return pl.pallas_call(
        _edge_T_body,
        grid=(Q_GRID,),
        in_specs=[
            pl.BlockSpec((EBLK, HD), hmap(0)),
            pl.BlockSpec((EBLK, HD), hmap(1)),
            pl.BlockSpec((EBLK, HD), hmap(2)),
            pl.BlockSpec((EBLK, HD), hmap(3)),
            pl.BlockSpec((HD, HD), lambda i: (0, 0)),
            pl.BlockSpec((1, HD), lambda i: (0, 0)),
        ],
        out_specs=pl.BlockSpec((EBLK, 4 * HD), lambda i: (i, 0)),
        out_shape=jax.ShapeDtypeStruct((Q, 4 * HD), F32),
    )(h_msg, h_msg, h_msg, h_msg, Wp, bp)


def _y_msg_body(hm_ref, W4_ref, b4_ref, m_ref, y_ref):
    hm = hm_ref[...]
    for g in range(G4):
        m_ref[g, :, :] = hm[:, g * HD:(g + 1) * HD]
    y = lax.dot_general(W4_ref[...], hm, (((1,), (1,)), ((), ())),
                        preferred_element_type=F32)
    y_ref[0] = y + b4_ref[...]


def _y_msg(hm4, W4, b4):
    return pl.pallas_call(
        _y_msg_body,
        grid=(Q_GRID,),
        in_specs=[
            pl.BlockSpec((EBLK, 4 * HD), lambda i: (i, 0)),
            pl.BlockSpec((8, 4 * HD), lambda i: (0, 0)),
            pl.BlockSpec((8, 1), lambda i: (0, 0)),
        ],
        out_specs=[
            pl.BlockSpec((G4, EBLK, HD), lambda i: (0, i, 0)),
            pl.BlockSpec((1, 8, EBLK), lambda i: (i, 0, 0)),
        ],
        out_shape=[
            jax.ShapeDtypeStruct((G4, Q, HD), F32),
            jax.ShapeDtypeStruct((Q_GRID, 8, EBLK), F32),
        ],
    )(hm4, W4, b4)


def _node_fin_body(h_ref, p0_ref, p1_ref, x_ref, Wu1_ref, Wu2_ref, bu_ref,
                   Wb1_ref, Wb2_ref, bb_ref, y_ref):
    agg = p0_ref[...].astype(F32) + p1_ref[...].astype(F32)
    hn = jnp.dot(h_ref[...], Wu1_ref[...], preferred_element_type=F32)
    hn = hn + jnp.dot(agg, Wu2_ref[...], preferred_element_type=F32) + bu_ref[...]
    hn = jnp.maximum(hn, 0.0)
    y = jnp.dot(hn, Wb1_ref[...], preferred_element_type=F32)
    y_ref[...] = y + jnp.dot(x_ref[...], Wb2_ref[...], preferred_element_type=F32) + bb_ref[...]


def _node_fin(h_node, p0, p1, x, Wu1, Wu2, bu, Wb1, Wb2, bb):
    const = pl.BlockSpec((HD, HD), lambda i: (0, 0))
    return pl.pallas_call(
        _node_fin_body,
        grid=(N_GRID,),
        in_specs=[
            pl.BlockSpec((NBLK, HD), lambda i: (i, 0)),
            pl.BlockSpec((NBLK, HD), lambda i: (i, 0)),
            pl.BlockSpec((NBLK, HD), lambda i: (i, 0)),
            pl.BlockSpec((NBLK, 3), lambda i: (i, 0)),
            const, const,
            pl.BlockSpec((1, HD), lambda i: (0, 0)),
            pl.BlockSpec((HD, 3), lambda i: (0, 0)),
            pl.BlockSpec((3, 3), lambda i: (0, 0)),
            pl.BlockSpec((1, 3), lambda i: (0, 0)),
        ],
        out_specs=pl.BlockSpec((NBLK, 3), lambda i: (i, 0)),
        out_shape=jax.ShapeDtypeStruct((N, 3), F32),
    )(h_node, p0, p1, x, Wu1, Wu2, bu, Wb1, Wb2, bb)


RCH = 160
CHE = RCH * G4
JW = 80
NCHUNK = Q // RCH
NC_SC = 2
NWORK = 16 * NC_SC
CPW_LO = NCHUNK // NWORK
CPW_XT = NCHUNK % NWORK
NPAD = 50048
NPS = NPAD // 16


def _sc_edge_body(T_h, src_h, dst_h, A_h, B_h, hm_h, part_h,
                  trows, arows, brows, hrows, idx_s, idx_d, agg, semT, semG):
    cid = lax.axis_index("c")
    sid = lax.axis_index("s")
    wid = cid * 16 + sid

    def _zrow(i, carry):
        hrows[i, ...] = jnp.zeros((HD,), jnp.bfloat16)
        return carry
    lax.fori_loop(0, CHE, _zrow, 0)
    zbase = sid * NPS
    for k in range(NPS // CHE):
        pltpu.sync_copy(hrows, agg.at[pl.ds(zbase + k * CHE, CHE)])
    rem = NPS % CHE
    if rem:
        pltpu.sync_copy(hrows.at[pl.ds(0, rem)],
                        agg.at[pl.ds(zbase + (NPS // CHE) * CHE, rem)])
    plsc.subcore_barrier()

    start = wid * CPW_LO + jnp.minimum(wid, CPW_XT)
    count = CPW_LO + (wid < CPW_XT).astype(jnp.int32)

    def _chunk(k, carry):
        c = start + k
        rbase = c * RCH
        for g in range(G4):
            for j in range(2):
                off = g * Q + rbase + j * JW
                pltpu.sync_copy(src_h.at[pl.ds(off, JW)], idx_s.at[2 * g + j])
                pltpu.sync_copy(dst_h.at[pl.ds(off, JW)], idx_d.at[2 * g + j])
        cpT = pltpu.async_copy(T_h.at[pl.ds(rbase, RCH)], trows, semT)
        cps = []
        for g in range(G4):
            for j in range(2):
                p0 = g * RCH + j * JW
                cps.append(pltpu.async_copy(
                    A_h.at[idx_s.at[2 * g + j]], arows.at[pl.ds(p0, JW)], semG))
                cps.append(pltpu.async_copy(
                    B_h.at[idx_d.at[2 * g + j]], brows.at[pl.ds(p0, JW)], semG))
        cpT.wait()
        for cp in cps:
            cp.wait()

        def _row(i, carry2):
            for g in range(G4):
                halves = []
                for h in (0, 16):
                    v = trows[i, pl.ds(g * HD + h, 16)] \
                        + arows[g * RCH + i, pl.ds(h, 16)] \
                        + brows[g * RCH + i, pl.ds(h, 16)]
                    r = jnp.maximum(v, 0.0)
                    trows[i, pl.ds(g * HD + h, 16)] = r
                    halves.append(r)
                hrows[g * RCH + i, ...] = plsc.pack(
                    halves[0], halves[1], format=plsc.PackFormat.INTERLEAVED)
            return carry2
        lax.fori_loop(0, RCH, _row, 0)

        for g in range(G4):
            for j in range(2):
                p0 = g * RCH + j * JW
                pltpu.sync_copy(hrows.at[pl.ds(p0, JW)],
                                agg.at[idx_d.at[2 * g + j]], add=True)
        pltpu.sync_copy(trows, hm_h.at[pl.ds(rbase, RCH)])
        return carry
    lax.fori_loop(0, count, _chunk, 0)

    plsc.subcore_barrier()
    pltpu.sync_copy(agg.at[pl.ds(sid * NPS, NPS)],
                    part_h.at[cid, pl.ds(sid * NPS, NPS)])


@functools.partial(
    pl.kernel,
    out_type=[jax.ShapeDtypeStruct((Q, 4 * HD), F32),
              jax.ShapeDtypeStruct((NC_SC, NPAD, HD), jnp.bfloat16)],
    mesh=plsc.VectorSubcoreMesh(core_axis_name="c", subcore_axis_name="s",
                                num_cores=NC_SC),
    compiler_params=pltpu.CompilerParams(use_tc_tiling_on_sc=False,
                                         needs_layout_passes=False),
    scratch_types=[
        pltpu.VMEM((RCH, 4 * HD), F32),
        pltpu.VMEM((CHE, HD), F32),
        pltpu.VMEM((CHE, HD), F32),
        pltpu.VMEM((CHE, HD), jnp.bfloat16),
        pltpu.VMEM((2 * G4, JW), jnp.int32),
        pltpu.VMEM((2 * G4, JW), jnp.int32),
        pltpu.VMEM_SHARED((NPAD, HD), jnp.bfloat16),
        pltpu.SemaphoreType.DMA,
        pltpu.SemaphoreType.DMA,
    ],
)
def _sc_edge(T_h, src_h, dst_h, A_h, B_h, hm_h, part_h,
             trows, arows, brows, hrows, idx_s, idx_d, agg, semT, semG):
    _sc_edge_body(T_h, src_h, dst_h, A_h, B_h, hm_h, part_h,
                  trows, arows, brows, hrows, idx_s, idx_d, agg, semT, semG)


def kernel(x, edge_index, h_msg, W_in, b_in, W_enc, b_enc, W_msg, b_msg,
           W_upd, b_upd, W_dec, b_dec, W_bel, b_bel):
    W1 = W_msg[0:HD]
    W2 = W_msg[HD:2 * HD]
    W3 = W_msg[2 * HD:3 * HD]
    b_in2 = b_in.reshape(1, HD)
    b_enc2 = b_enc.reshape(1, HD)
    b_msg2 = b_msg.reshape(1, HD)
    b_upd2 = b_upd.reshape(1, HD)
    b_bel2 = b_bel.reshape(1, 3)
    Wu1 = W_upd[0:HD]
    Wu2 = W_upd[HD:2 * HD]
    Wb1 = W_bel[0:HD]
    Wb2 = W_bel[HD:HD + 3]

    h_node, A, B, Wp, bp = _node_pre(x, W_in, b_in2, W1, W2, W_enc, W3,
                                     b_enc2, b_msg2)
    T4 = _edge_T(h_msg, Wp, bp)

    src1 = edge_index[0]
    dst1 = edge_index[1]
    hm4, parts = _sc_edge(T4, src1, dst1, A, B)

    p0 = parts[0, :N]
    p1 = jnp.zeros_like(p0) if NC_SC == 1 else parts[1, :N]
    cols = jnp.arange(HD)
    rho = jnp.where(cols % 2 == 0, cols // 2, HD // 2 + cols // 2)
    Wu2 = Wu2[rho]

    W4 = jnp.kron(jnp.eye(G4, dtype=F32), W_dec.T)
    b4 = jnp.tile(b_dec, G4).reshape(8, 1)
    hmg, yT = _y_msg(hm4, W4, b4)
    h_msg_new = hmg.reshape(E, HD)
    y_msg = (yT.transpose(1, 0, 2).reshape(G4, 2, Q)
             .transpose(0, 2, 1).reshape(E, 2))

    y_beliefs = _node_fin(h_node, p0, p1, x, Wu1, Wu2, b_upd2, Wb1, Wb2, b_bel2)
    return (h_msg_new, y_msg, y_beliefs)


# speedup vs baseline: 3.7058x; 1.2028x over previous
import functools

import jax
import jax.numpy as jnp
from jax import lax
from jax.experimental import pallas as pl
from jax.experimental.pallas import tpu as pltpu
from jax.experimental.pallas import tpu_sc as plsc

N = 50000
E = 800000
HD = 32
Q = E // 4
G4 = 4

NBLK = 2000
EBLK = 2000
N_GRID = N // NBLK
Q_GRID = Q // EBLK

F32 = jnp.float32


def _node_pre_body(x_ref, W_in_ref, b_in_ref, W1_ref, W2_ref, Wenc_ref,
                   W3_ref, benc_ref, bmsg_ref,
                   h_ref, A_ref, B_ref, Wp_ref, bp_ref):
    h = jnp.dot(x_ref[...], W_in_ref[...], preferred_element_type=F32) + b_in_ref[...]
    h_ref[...] = h
    A_ref[...] = jnp.dot(h, W1_ref[...], preferred_element_type=F32)
    B_ref[...] = jnp.dot(h, W2_ref[...], preferred_element_type=F32)

    @pl.when(pl.program_id(0) == 0)
    def _():
        Wp_ref[...] = jnp.dot(Wenc_ref[...], W3_ref[...], preferred_element_type=F32)
        bp_ref[...] = jnp.dot(benc_ref[...], W3_ref[...], preferred_element_type=F32) + bmsg_ref[...]


def _node_pre(x, W_in, b_in, W1, W2, W_enc, W3, b_enc, b_msg):
    const = pl.BlockSpec((HD, HD), lambda i: (0, 0))
    constb = pl.BlockSpec((1, HD), lambda i: (0, 0))
    return pl.pallas_call(
        _node_pre_body,
        grid=(N_GRID,),
        in_specs=[
            pl.BlockSpec((NBLK, 3), lambda i: (i, 0)),
            pl.BlockSpec((3, HD), lambda i: (0, 0)),
            constb, const, const, const, const, constb, constb,
        ],
        out_specs=[
            pl.BlockSpec((NBLK, HD), lambda i: (i, 0)),
            pl.BlockSpec((NBLK, HD), lambda i: (i, 0)),
            pl.BlockSpec((NBLK, HD), lambda i: (i, 0)),
            pl.BlockSpec((HD, HD), lambda i: (0, 0)),
            pl.BlockSpec((1, HD), lambda i: (0, 0)),
        ],
        out_shape=[
            jax.ShapeDtypeStruct((N, HD), F32),
            jax.ShapeDtypeStruct((N, HD), F32),
            jax.ShapeDtypeStruct((N, HD), F32),
            jax.ShapeDtypeStruct((HD, HD), F32),
            jax.ShapeDtypeStruct((1, HD), F32),
        ],
    )(x, W_in, b_in, W1, W2, W_enc, W3, b_enc, b_msg)


def _edge_T_body(h0_ref, h1_ref, h2_ref, h3_ref, Wp_ref, bp_ref, T_ref):
    Wp = Wp_ref[...]
    bp = bp_ref[...]
    parts = [jnp.dot(h_ref[...], Wp, preferred_element_type=F32) + bp
             for h_ref in (h0_ref, h1_ref, h2_ref, h3_ref)]
    T_ref[...] = jnp.concatenate(parts, axis=1)


def _edge_T(h_msg, Wp, bp):
    def hmap(g):
        return lambda i: (g * Q_GRID + i, 0)
    return pl.pallas_call(
        _edge_T_body,
        grid=(Q_GRID,),
        in_specs=[
            pl.BlockSpec((EBLK, HD), hmap(0)),
            pl.BlockSpec((EBLK, HD), hmap(1)),
            pl.BlockSpec((EBLK, HD), hmap(2)),
            pl.BlockSpec((EBLK, HD), hmap(3)),
            pl.BlockSpec((HD, HD), lambda i: (0, 0)),
            pl.BlockSpec((1, HD), lambda i: (0, 0)),
        ],
        out_specs=pl.BlockSpec((EBLK, 4 * HD), lambda i: (i, 0)),
        out_shape=jax.ShapeDtypeStruct((QP, 4 * HD), F32),
    )(h_msg, h_msg, h_msg, h_msg, Wp, bp)


def _y_msg_body(hm_ref, W4_ref, b4_ref, m_ref, y_ref):
    hm = hm_ref[...]
    for g in range(G4):
        m_ref[g, :, :] = hm[:, g * HD:(g + 1) * HD]
    y = lax.dot_general(W4_ref[...], hm, (((1,), (1,)), ((), ())),
                        preferred_element_type=F32)
    y_ref[0] = y + b4_ref[...]


def _y_msg(hm4, W4, b4):
    return pl.pallas_call(
        _y_msg_body,
        grid=(Q_GRID,),
        in_specs=[
            pl.BlockSpec((EBLK, 4 * HD), lambda i: (i, 0)),
            pl.BlockSpec((8, 4 * HD), lambda i: (0, 0)),
            pl.BlockSpec((8, 1), lambda i: (0, 0)),
        ],
        out_specs=[
            pl.BlockSpec((G4, EBLK, HD), lambda i: (0, i, 0)),
            pl.BlockSpec((1, 8, EBLK), lambda i: (i, 0, 0)),
        ],
        out_shape=[
            jax.ShapeDtypeStruct((G4, Q, HD), F32),
            jax.ShapeDtypeStruct((Q_GRID, 8, EBLK), F32),
        ],
    )(hm4, W4, b4)


def _node_fin_body(h_ref, p0_ref, p1_ref, x_ref, Wu1_ref, Wu2_ref, bu_ref,
                   Wb1_ref, Wb2_ref, bb_ref, y_ref):
    agg = p0_ref[...].astype(F32) + p1_ref[...].astype(F32)
    hn = jnp.dot(h_ref[...], Wu1_ref[...], preferred_element_type=F32)
    hn = hn + jnp.dot(agg, Wu2_ref[...], preferred_element_type=F32) + bu_ref[...]
    hn = jnp.maximum(hn, 0.0)
    y = jnp.dot(hn, Wb1_ref[...], preferred_element_type=F32)
    y_ref[...] = y + jnp.dot(x_ref[...], Wb2_ref[...], preferred_element_type=F32) + bb_ref[...]


def _node_fin(h_node, p0, p1, x, Wu1, Wu2, bu, Wb1, Wb2, bb):
    const = pl.BlockSpec((HD, HD), lambda i: (0, 0))
    return pl.pallas_call(
        _node_fin_body,
        grid=(N_GRID,),
        in_specs=[
            pl.BlockSpec((NBLK, HD), lambda i: (i, 0)),
            pl.BlockSpec((NBLK, HD), lambda i: (i, 0)),
            pl.BlockSpec((NBLK, HD), lambda i: (i, 0)),
            pl.BlockSpec((NBLK, 3), lambda i: (i, 0)),
            const, const,
            pl.BlockSpec((1, HD), lambda i: (0, 0)),
            pl.BlockSpec((HD, 3), lambda i: (0, 0)),
            pl.BlockSpec((3, 3), lambda i: (0, 0)),
            pl.BlockSpec((1, 3), lambda i: (0, 0)),
        ],
        out_specs=pl.BlockSpec((NBLK, 3), lambda i: (i, 0)),
        out_shape=jax.ShapeDtypeStruct((N, 3), F32),
    )(h_node, p0, p1, x, Wu1, Wu2, bu, Wb1, Wb2, bb)


QP = 204800
RCH = 64
CHE = RCH * G4
JW = RCH
NCHUNK = QP // RCH
NC_SC = 2
NWORK = 16 * NC_SC
CPW = NCHUNK // NWORK
NPAD = 50048
NPS = NPAD // 16
BF16 = jnp.bfloat16


def _sc_edge_body(T_h, src_h, dst_h, A_h, B_h, hm_h, part_h,
                  tin, tout, arow, brow, hrow, isx, idx, agg,
                  semG0, semG1, semS0, semS1, semI0, semI1):
    cid = lax.axis_index("c")
    sid = lax.axis_index("s")
    wid = cid * 16 + sid
    start = wid * CPW
    semG = (semG0, semG1)
    semS = (semS0, semS1)
    semI = (semI0, semI1)

    def _zrow(i, carry):
        hrow[0, i, ...] = jnp.zeros((HD,), BF16)
        return carry
    lax.fori_loop(0, CHE, _zrow, 0)
    zbase = sid * NPS
    for k in range(NPS // CHE):
        pltpu.sync_copy(hrow.at[0], agg.at[pl.ds(zbase + k * CHE, CHE)])
    rem = NPS % CHE
    if rem:
        pltpu.sync_copy(hrow.at[0, pl.ds(0, rem)],
                        agg.at[pl.ds(zbase + (NPS // CHE) * CHE, rem)])
    plsc.subcore_barrier()

    def idx_copies(c, b):
        cbase = jnp.minimum(c * RCH, Q - RCH)
        cps = []
        for g in range(G4):
            off = g * Q + cbase
            cps.append(pltpu.make_async_copy(
                src_h.at[pl.ds(off, JW)], isx.at[b, g], semI[b]))
            cps.append(pltpu.make_async_copy(
                dst_h.at[pl.ds(off, JW)], idx.at[b, g], semI[b]))
        return cps

    def _fix_pad(c, b):
        @pl.when(c * RCH + RCH > Q)
        def _():
            for g in range(G4):
                for j in range(JW // 16):
                    rowv = c * RCH + 16 * j + jnp.arange(16, dtype=jnp.int32)
                    v = idx[b, g, pl.ds(16 * j, 16)]
                    t = N + (g * (JW // 16) + j) % (NPAD - 8 - N)
                    idx[b, g, pl.ds(16 * j, 16)] = jnp.where(rowv >= Q, t, v)

    def gath_copies(c, b):
        cps = [pltpu.make_async_copy(T_h.at[pl.ds(c * RCH, RCH)],
                                     tin.at[b], semG[b])]
        for g in range(G4):
            cps.append(pltpu.make_async_copy(
                A_h.at[isx.at[b, g]], arow.at[b, pl.ds(g * RCH, JW)], semG[b]))
            cps.append(pltpu.make_async_copy(
                B_h.at[idx.at[b, g]], brow.at[b, pl.ds(g * RCH, JW)], semG[b]))
        return cps

    def scat_copies(c, b):
        adds = [pltpu.make_async_copy(
            hrow.at[b, pl.ds(g * RCH, JW)], agg.at[idx.at[b, g]], semS[b])
            for g in range(G4)]
        out = pltpu.make_async_copy(tout.at[b],
                                    hm_h.at[pl.ds(c * RCH, RCH)], semS[b])
        return adds, out

    def compute(b):
        def _row(i, carry):
            for g in range(G4):
                tl = tin[b, i, pl.ds(g * HD, 16)]
                th = tin[b, i, pl.ds(g * HD + 16, 16)]
                al = arow[b, g * RCH + i, pl.ds(0, 16)]
                ah = arow[b, g * RCH + i, pl.ds(16, 16)]
                bl = brow[b, g * RCH + i, pl.ds(0, 16)]
                bh = brow[b, g * RCH + i, pl.ds(16, 16)]
                rl = jnp.maximum(tl + al + bl, 0.0)
                rh = jnp.maximum(th + ah + bh, 0.0)
                tout[b, i, pl.ds(g * HD, 16)] = rl
                tout[b, i, pl.ds(g * HD + 16, 16)] = rh
                hrow[b, g * RCH + i, ...] = plsc.pack(
                    rl, rh, format=plsc.PackFormat.INTERLEAVED)
            return carry
        lax.fori_loop(0, RCH, _row, 0)

    def _pair(k, carry):
        c0 = start + 2 * k
        c1 = c0 + 1
        iA = idx_copies(c0, 0)
        iB = idx_copies(c1, 1)
        for cp in iA + iB:
            cp.start()
        for cp in iA:
            cp.wait()
        gA = gath_copies(c0, 0)
        for cp in gA:
            cp.start()
        for cp in iB:
            cp.wait()
        gB = gath_copies(c1, 1)
        for cp in gB:
            cp.start()
        for cp in gA:
            cp.wait()
        _fix_pad(c0, 0)
        compute(0)
        for cp in gB:
            cp.wait()
        _fix_pad(c1, 1)
        sA, oA = scat_copies(c0, 0)
        for cp in sA:
            cp.start(add=True)
            cp.wait()
        oA.start()
        oA.wait()
        compute(1)
        sB, oB = scat_copies(c1, 1)
        for cp in sB:
            cp.start(add=True)
            cp.wait()
        oB.start()
        oB.wait()
        return carry
    lax.fori_loop(0, CPW // 2, _pair, 0)

    plsc.subcore_barrier()
    pltpu.sync_copy(agg.at[pl.ds(sid * NPS, NPS)],
                    part_h.at[cid, pl.ds(sid * NPS, NPS)])


@functools.partial(
    pl.kernel,
    out_type=[jax.ShapeDtypeStruct((QP, 4 * HD), F32),
              jax.ShapeDtypeStruct((NC_SC, NPAD, HD), jnp.bfloat16)],
    mesh=plsc.VectorSubcoreMesh(core_axis_name="c", subcore_axis_name="s",
                                num_cores=NC_SC),
    compiler_params=pltpu.CompilerParams(use_tc_tiling_on_sc=False,
                                         needs_layout_passes=False),
    scratch_types=[
        pltpu.VMEM((2, RCH, 4 * HD), F32),
        pltpu.VMEM((2, RCH, 4 * HD), F32),
        pltpu.VMEM((2, CHE, HD), F32),
        pltpu.VMEM((2, CHE, HD), F32),
        pltpu.VMEM((2, CHE, HD), BF16),
        pltpu.VMEM((2, G4, JW), jnp.int32),
        pltpu.VMEM((2, G4, JW), jnp.int32),
        pltpu.VMEM_SHARED((NPAD, HD), BF16),
        pltpu.SemaphoreType.DMA,
        pltpu.SemaphoreType.DMA,
        pltpu.SemaphoreType.DMA,
        pltpu.SemaphoreType.DMA,
        pltpu.SemaphoreType.DMA,
        pltpu.SemaphoreType.DMA,
    ],
)
def _sc_edge(T_h, src_h, dst_h, A_h, B_h, hm_h, part_h,
             tin, tout, arow, brow, hrow, isx, idx, agg,
             semG0, semG1, semS0, semS1, semI0, semI1):
    _sc_edge_body(T_h, src_h, dst_h, A_h, B_h, hm_h, part_h,
                  tin, tout, arow, brow, hrow, isx, idx, agg,
                  semG0, semG1, semS0, semS1, semI0, semI1)


def kernel(x, edge_index, h_msg, W_in, b_in, W_enc, b_enc, W_msg, b_msg,
           W_upd, b_upd, W_dec, b_dec, W_bel, b_bel):
    W1 = W_msg[0:HD]
    W2 = W_msg[HD:2 * HD]
    W3 = W_msg[2 * HD:3 * HD]
    b_in2 = b_in.reshape(1, HD)
    b_enc2 = b_enc.reshape(1, HD)
    b_msg2 = b_msg.reshape(1, HD)
    b_upd2 = b_upd.reshape(1, HD)
    b_bel2 = b_bel.reshape(1, 3)
    Wu1 = W_upd[0:HD]
    Wu2 = W_upd[HD:2 * HD]
    Wb1 = W_bel[0:HD]
    Wb2 = W_bel[HD:HD + 3]

    cols = jnp.arange(HD)
    rho = jnp.where(cols % 2 == 0, cols // 2, HD // 2 + cols // 2)
    h_node, A, B, Wp, bp = _node_pre(x, W_in, b_in2, W1, W2,
                                     W_enc, W3, b_enc2, b_msg2)
    T4 = _edge_T(h_msg, Wp, bp)

    src1 = edge_index[0]
    dst1 = edge_index[1]
    hm4, parts = _sc_edge(T4, src1, dst1, A, B)

    p0 = parts[0, :N]
    p1 = jnp.zeros_like(p0) if NC_SC == 1 else parts[1, :N]
    Wu2 = Wu2[rho]

    W4 = jnp.kron(jnp.eye(G4, dtype=F32), W_dec.T)
    b4 = jnp.tile(b_dec, G4).reshape(8, 1)
    hmg, yT = _y_msg(hm4[:Q], W4, b4)
    h_msg_new = hmg.reshape(E, HD)
    y_msg = (yT.transpose(1, 0, 2).reshape(G4, 2, Q)
             .transpose(0, 2, 1).reshape(E, 2))

    y_beliefs = _node_fin(h_node, p0, p1, x, Wu1, Wu2, b_upd2, Wb1, Wb2, b_bel2)
    return (h_msg_new, y_msg, y_beliefs)


# speedup vs baseline: 4.2249x; 1.1401x over previous
import functools

import jax
import jax.numpy as jnp
from jax import lax
from jax.experimental import pallas as pl
from jax.experimental.pallas import tpu as pltpu
from jax.experimental.pallas import tpu_sc as plsc

N = 50000
E = 800000
HD = 32
Q = E // 4
G4 = 4

NBLK = 2000
EBLK = 2000
N_GRID = N // NBLK
Q_GRID = Q // EBLK

F32 = jnp.float32


def _node_pre_body(x_ref, W_in_ref, b_in_ref, W1_ref, W2_ref, Wenc_ref,
                   W3_ref, benc_ref, bmsg_ref,
                   h_ref, A_ref, B_ref, Wp_ref, bp_ref):
    h = jnp.dot(x_ref[...], W_in_ref[...], preferred_element_type=F32) + b_in_ref[...]
    h_ref[...] = h
    A_ref[...] = jnp.dot(h, W1_ref[...], preferred_element_type=F32).astype(jnp.bfloat16)
    B_ref[...] = jnp.dot(h, W2_ref[...], preferred_element_type=F32).astype(jnp.bfloat16)

    @pl.when(pl.program_id(0) == 0)
    def _():
        Wp_ref[...] = jnp.dot(Wenc_ref[...], W3_ref[...], preferred_element_type=F32)
        bp_ref[...] = jnp.dot(benc_ref[...], W3_ref[...], preferred_element_type=F32) + bmsg_ref[...]


def _node_pre(x, W_in, b_in, W1, W2, W_enc, W3, b_enc, b_msg):
    const = pl.BlockSpec((HD, HD), lambda i: (0, 0))
    constb = pl.BlockSpec((1, HD), lambda i: (0, 0))
    return pl.pallas_call(
        _node_pre_body,
        grid=(N_GRID,),
        in_specs=[
            pl.BlockSpec((NBLK, 3), lambda i: (i, 0)),
            pl.BlockSpec((3, HD), lambda i: (0, 0)),
            constb, const, const, const, const, constb, constb,
        ],
        out_specs=[
            pl.BlockSpec((NBLK, HD), lambda i: (i, 0)),
            pl.BlockSpec((NBLK, HD), lambda i: (i, 0)),
            pl.BlockSpec((NBLK, HD), lambda i: (i, 0)),
            pl.BlockSpec((HD, HD), lambda i: (0, 0)),
            pl.BlockSpec((1, HD), lambda i: (0, 0)),
        ],
        out_shape=[
            jax.ShapeDtypeStruct((N, HD), F32),
            jax.ShapeDtypeStruct((N, HD), jnp.bfloat16),
            jax.ShapeDtypeStruct((N, HD), jnp.bfloat16),
            jax.ShapeDtypeStruct((HD, HD), F32),
            jax.ShapeDtypeStruct((1, HD), F32),
        ],
    )(x, W_in, b_in, W1, W2, W_enc, W3, b_enc, b_msg)


def _edge_T_body(h0_ref, h1_ref, h2_ref, h3_ref, Wp_ref, bp_ref, T_ref):
    Wp = Wp_ref[...]
    bp = bp_ref[...]
    parts = [jnp.dot(h_ref[...], Wp, preferred_element_type=F32) + bp
             for h_ref in (h0_ref, h1_ref, h2_ref, h3_ref)]
    T_ref[...] = jnp.concatenate(parts, axis=1)


def _edge_T(h_msg, Wp, bp):
    def hmap(g):
        return lambda i: (g * Q_GRID + i, 0)
    return pl.pallas_call(
        _edge_T_body,
        grid=(Q_GRID,),
        in_specs=[
            pl.BlockSpec((EBLK, HD), hmap(0)),
            pl.BlockSpec((EBLK, HD), hmap(1)),
            pl.BlockSpec((EBLK, HD), hmap(2)),
            pl.BlockSpec((EBLK, HD), hmap(3)),
            pl.BlockSpec((HD, HD), lambda i: (0, 0)),
            pl.BlockSpec((1, HD), lambda i: (0, 0)),
        ],
        out_specs=pl.BlockSpec((EBLK, 4 * HD), lambda i: (i, 0)),
        out_shape=jax.ShapeDtypeStruct((QP, 4 * HD), F32),
    )(h_msg, h_msg, h_msg, h_msg, Wp, bp)


def _y_msg_body(hm_ref, W4_ref, b4_ref, m_ref, y_ref):
    hm = hm_ref[...]
    for g in range(G4):
        m_ref[g, :, :] = hm[:, g * HD:(g + 1) * HD]
    y = lax.dot_general(W4_ref[...], hm, (((1,), (1,)), ((), ())),
                        preferred_element_type=F32)
    y_ref[0] = y + b4_ref[...]


def _y_msg(hm4, W4, b4):
    return pl.pallas_call(
        _y_msg_body,
        grid=(Q_GRID,),
        in_specs=[
            pl.BlockSpec((EBLK, 4 * HD), lambda i: (i, 0)),
            pl.BlockSpec((8, 4 * HD), lambda i: (0, 0)),
            pl.BlockSpec((8, 1), lambda i: (0, 0)),
        ],
        out_specs=[
            pl.BlockSpec((G4, EBLK, HD), lambda i: (0, i, 0)),
            pl.BlockSpec((1, 8, EBLK), lambda i: (i, 0, 0)),
        ],
        out_shape=[
            jax.ShapeDtypeStruct((G4, Q, HD), F32),
            jax.ShapeDtypeStruct((Q_GRID, 8, EBLK), F32),
        ],
    )(hm4, W4, b4)


def _node_fin_body(h_ref, p0_ref, p1_ref, x_ref, Wu1_ref, Wu2_ref, bu_ref,
                   Wb1_ref, Wb2_ref, bb_ref, y_ref):
    agg = p0_ref[...].astype(F32) + p1_ref[...].astype(F32)
    hn = jnp.dot(h_ref[...], Wu1_ref[...], preferred_element_type=F32)
    hn = hn + jnp.dot(agg, Wu2_ref[...], preferred_element_type=F32) + bu_ref[...]
    hn = jnp.maximum(hn, 0.0)
    y = jnp.dot(hn, Wb1_ref[...], preferred_element_type=F32)
    y_ref[...] = y + jnp.dot(x_ref[...], Wb2_ref[...], preferred_element_type=F32) + bb_ref[...]


def _node_fin(h_node, p0, p1, x, Wu1, Wu2, bu, Wb1, Wb2, bb):
    const = pl.BlockSpec((HD, HD), lambda i: (0, 0))
    return pl.pallas_call(
        _node_fin_body,
        grid=(N_GRID,),
        in_specs=[
            pl.BlockSpec((NBLK, HD), lambda i: (i, 0)),
            pl.BlockSpec((NBLK, HD), lambda i: (i, 0)),
            pl.BlockSpec((NBLK, HD), lambda i: (i, 0)),
            pl.BlockSpec((NBLK, 3), lambda i: (i, 0)),
            const, const,
            pl.BlockSpec((1, HD), lambda i: (0, 0)),
            pl.BlockSpec((HD, 3), lambda i: (0, 0)),
            pl.BlockSpec((3, 3), lambda i: (0, 0)),
            pl.BlockSpec((1, 3), lambda i: (0, 0)),
        ],
        out_specs=pl.BlockSpec((NBLK, 3), lambda i: (i, 0)),
        out_shape=jax.ShapeDtypeStruct((N, 3), F32),
    )(h_node, p0, p1, x, Wu1, Wu2, bu, Wb1, Wb2, bb)


QP = 204800
RCH = 64
CHE = RCH * G4
JW = RCH
NCHUNK = QP // RCH
NC_SC = 2
NWORK = 16 * NC_SC
CPW = NCHUNK // NWORK
NPAD = 50048
NPS = NPAD // 16
BF16 = jnp.bfloat16


def _sc_edge_body(T_h, src_h, dst_h, A_h, B_h, hm_h, part_h,
                  tin, tout, arow, brow, hrow, isx, idx, agg,
                  semG0, semG1, semS0, semS1, semI0, semI1):
    cid = lax.axis_index("c")
    sid = lax.axis_index("s")
    wid = cid * 16 + sid
    start = wid * CPW
    semG = (semG0, semG1)
    semS = (semS0, semS1)
    semI = (semI0, semI1)

    def _zrow(i, carry):
        hrow[0, i, ...] = jnp.zeros((HD,), BF16)
        return carry
    lax.fori_loop(0, CHE, _zrow, 0)
    zbase = sid * NPS
    for k in range(NPS // CHE):
        pltpu.sync_copy(hrow.at[0], agg.at[pl.ds(zbase + k * CHE, CHE)])
    rem = NPS % CHE
    if rem:
        pltpu.sync_copy(hrow.at[0, pl.ds(0, rem)],
                        agg.at[pl.ds(zbase + (NPS // CHE) * CHE, rem)])
    plsc.subcore_barrier()

    def idx_copies(c, b):
        cbase = jnp.minimum(c * RCH, Q - RCH)
        cps = []
        for g in range(G4):
            off = g * Q + cbase
            cps.append(pltpu.make_async_copy(
                src_h.at[pl.ds(off, JW)], isx.at[b, g], semI[b]))
            cps.append(pltpu.make_async_copy(
                dst_h.at[pl.ds(off, JW)], idx.at[b, g], semI[b]))
        return cps

    def _fix_pad(c, b):
        @pl.when(c * RCH + RCH > Q)
        def _():
            for g in range(G4):
                for j in range(JW // 16):
                    rowv = c * RCH + 16 * j + jnp.arange(16, dtype=jnp.int32)
                    v = idx[b, g, pl.ds(16 * j, 16)]
                    t = N + (g * (JW // 16) + j) % (NPAD - 8 - N)
                    idx[b, g, pl.ds(16 * j, 16)] = jnp.where(rowv >= Q, t, v)

    def gath_copies(c, b):
        cps = [pltpu.make_async_copy(T_h.at[pl.ds(c * RCH, RCH)],
                                     tin.at[b], semG[b])]
        for g in range(G4):
            cps.append(pltpu.make_async_copy(
                A_h.at[isx.at[b, g]], arow.at[b, pl.ds(g * RCH, JW)], semG[b]))
            cps.append(pltpu.make_async_copy(
                B_h.at[idx.at[b, g]], brow.at[b, pl.ds(g * RCH, JW)], semG[b]))
        return cps

    def scat_copies(c, b):
        adds = [pltpu.make_async_copy(
            hrow.at[b, pl.ds(g * RCH, JW)], agg.at[idx.at[b, g]], semS[b])
            for g in range(G4)]
        out = pltpu.make_async_copy(tout.at[b],
                                    hm_h.at[pl.ds(c * RCH, RCH)], semS[b])
        return adds, out

    def compute(b):
        def _row(i):
            for g in range(G4):
                tl = tin[b, i, pl.ds(g * HD, 16)]
                th = tin[b, i, pl.ds(g * HD + 16, 16)]
                al, ah = plsc.unpack(arow[b, g * RCH + i, ...],
                                     format=plsc.PackFormat.INTERLEAVED)
                bl, bh = plsc.unpack(brow[b, g * RCH + i, ...],
                                     format=plsc.PackFormat.INTERLEAVED)
                rl = jnp.maximum(tl + al + bl, 0.0)
                rh = jnp.maximum(th + ah + bh, 0.0)
                tout[b, i, pl.ds(g * HD, 16)] = rl
                tout[b, i, pl.ds(g * HD + 16, 16)] = rh
                hrow[b, g * RCH + i, ...] = plsc.pack(
                    rl, rh, format=plsc.PackFormat.INTERLEAVED)
        plsc.parallel_loop(0, RCH)(_row)

    def _pair(k, carry):
        c0 = start + 2 * k
        c1 = c0 + 1
        iA = idx_copies(c0, 0)
        iB = idx_copies(c1, 1)
        for cp in iA + iB:
            cp.start()
        for cp in iA:
            cp.wait()
        gA = gath_copies(c0, 0)
        for cp in gA:
            cp.start()
        for cp in iB:
            cp.wait()
        gB = gath_copies(c1, 1)
        for cp in gB:
            cp.start()
        for cp in gA:
            cp.wait()
        _fix_pad(c0, 0)
        compute(0)
        for cp in gB:
            cp.wait()
        _fix_pad(c1, 1)
        sA, oA = scat_copies(c0, 0)
        for cp in sA:
            cp.start(add=True)
            cp.wait()
        oA.start()
        oA.wait()
        compute(1)
        sB, oB = scat_copies(c1, 1)
        for cp in sB:
            cp.start(add=True)
            cp.wait()
        oB.start()
        oB.wait()
        return carry
    lax.fori_loop(0, CPW // 2, _pair, 0)

    plsc.subcore_barrier()
    pltpu.sync_copy(agg.at[pl.ds(sid * NPS, NPS)],
                    part_h.at[cid, pl.ds(sid * NPS, NPS)])


@functools.partial(
    pl.kernel,
    out_type=[jax.ShapeDtypeStruct((QP, 4 * HD), F32),
              jax.ShapeDtypeStruct((NC_SC, NPAD, HD), jnp.bfloat16)],
    mesh=plsc.VectorSubcoreMesh(core_axis_name="c", subcore_axis_name="s",
                                num_cores=NC_SC),
    compiler_params=pltpu.CompilerParams(use_tc_tiling_on_sc=False,
                                         needs_layout_passes=False),
    scratch_types=[
        pltpu.VMEM((2, RCH, 4 * HD), F32),
        pltpu.VMEM((2, RCH, 4 * HD), F32),
        pltpu.VMEM((2, CHE, HD), BF16),
        pltpu.VMEM((2, CHE, HD), BF16),
        pltpu.VMEM((2, CHE, HD), BF16),
        pltpu.VMEM((2, G4, JW), jnp.int32),
        pltpu.VMEM((2, G4, JW), jnp.int32),
        pltpu.VMEM_SHARED((NPAD, HD), BF16),
        pltpu.SemaphoreType.DMA,
        pltpu.SemaphoreType.DMA,
        pltpu.SemaphoreType.DMA,
        pltpu.SemaphoreType.DMA,
        pltpu.SemaphoreType.DMA,
        pltpu.SemaphoreType.DMA,
    ],
)
def _sc_edge(T_h, src_h, dst_h, A_h, B_h, hm_h, part_h,
             tin, tout, arow, brow, hrow, isx, idx, agg,
             semG0, semG1, semS0, semS1, semI0, semI1):
    _sc_edge_body(T_h, src_h, dst_h, A_h, B_h, hm_h, part_h,
                  tin, tout, arow, brow, hrow, isx, idx, agg,
                  semG0, semG1, semS0, semS1, semI0, semI1)


def kernel(x, edge_index, h_msg, W_in, b_in, W_enc, b_enc, W_msg, b_msg,
           W_upd, b_upd, W_dec, b_dec, W_bel, b_bel):
    W1 = W_msg[0:HD]
    W2 = W_msg[HD:2 * HD]
    W3 = W_msg[2 * HD:3 * HD]
    b_in2 = b_in.reshape(1, HD)
    b_enc2 = b_enc.reshape(1, HD)
    b_msg2 = b_msg.reshape(1, HD)
    b_upd2 = b_upd.reshape(1, HD)
    b_bel2 = b_bel.reshape(1, 3)
    Wu1 = W_upd[0:HD]
    Wu2 = W_upd[HD:2 * HD]
    Wb1 = W_bel[0:HD]
    Wb2 = W_bel[HD:HD + 3]

    cols = jnp.arange(HD)
    rho = jnp.where(cols % 2 == 0, cols // 2, HD // 2 + cols // 2)
    h_node, A, B, Wp, bp = _node_pre(x, W_in, b_in2, W1[:, rho], W2[:, rho],
                                     W_enc, W3, b_enc2, b_msg2)
    T4 = _edge_T(h_msg, Wp, bp)

    src1 = edge_index[0]
    dst1 = edge_index[1]
    hm4, parts = _sc_edge(T4, src1, dst1, A, B)

    p0 = parts[0, :N]
    p1 = jnp.zeros_like(p0) if NC_SC == 1 else parts[1, :N]
    Wu2 = Wu2[rho]

    W4 = jnp.kron(jnp.eye(G4, dtype=F32), W_dec.T)
    b4 = jnp.tile(b_dec, G4).reshape(8, 1)
    hmg, yT = _y_msg(hm4[:Q], W4, b4)
    h_msg_new = hmg.reshape(E, HD)
    y_msg = (yT.transpose(1, 0, 2).reshape(G4, 2, Q)
             .transpose(0, 2, 1).reshape(E, 2))

    y_beliefs = _node_fin(h_node, p0, p1, x, Wu1, Wu2, b_upd2, Wb1, Wb2, b_bel2)
    return (h_msg_new, y_msg, y_beliefs)
